# baseline (speedup 1.0000x reference)
import functools

import jax
import jax.numpy as jnp
from jax import lax
from jax.experimental import pallas as pl
from jax.experimental.pallas import tpu as pltpu
from jax.experimental.pallas import tpu_sc as plsc

B, C, H, W = 8, 4, 512, 512
HW = H * W
N = B * HW
GAMMA = 2.0
SMOOTH = 1e-05

SB_SC = 2
TB_TC = B - SB_SC

NC, NS, L = 2, 16, 16
NW = NC * NS
ROWS_W = SB_SC * H // NW
CHUNKR = 16
NCHUNK = ROWS_W // CHUNKR
NVEC = CHUNKR * W // L

HB_TC = 128
LN2 = 0.6931471805599453
K_SEL = 1.0 - SMOOTH - SMOOTH / (C - 1)
K_SUM = SMOOTH / (C - 1)


def _fast_log(x):
    bits = lax.bitcast_convert_type(x, jnp.int32)
    ef = jnp.right_shift(bits, 23).astype(jnp.float32)
    mbits = jnp.bitwise_or(jnp.bitwise_and(bits, 0x007FFFFF), 0x3F800000)
    m = lax.bitcast_convert_type(mbits, jnp.float32)
    z = (m - 1.0) / (m + 1.0)
    z2 = z * z
    p = z * (2.0 + z2 * (2.0 / 3.0 + z2 * (2.0 / 5.0 + z2 * (2.0 / 7.0))))
    return ef * LN2 + (p - 127.0 * LN2)


def _sc_body(lr_hbm, tr_hbm, out_hbm,
             lb0_0, lb0_1, lb0_2, lb0_3, lb1_0, lb1_1, lb1_2, lb1_3,
             tb0_1, tb0_2, tb0_3, tb1_1, tb1_2, tb1_3,
             accbuf, sem0, sem1):
    cid = lax.axis_index("c")
    sid = lax.axis_index("s")
    wid = cid * NS + sid
    lbuf = ((lb0_0, lb0_1, lb0_2, lb0_3), (lb1_0, lb1_1, lb1_2, lb1_3))
    tbuf = ((tb0_1, tb0_2, tb0_3), (tb1_1, tb1_2, tb1_3))
    sems = (sem0, sem1)

    def start(k, slot):
        g0 = wid * ROWS_W + k * CHUNKR
        b = g0 // H
        h0 = g0 % H
        cps = []
        for c in range(C):
            cps.append(pltpu.async_copy(
                lr_hbm.at[b * C + c, pl.ds(h0, CHUNKR), :],
                lbuf[slot][c], sems[slot]))
        for c in range(1, C):
            cps.append(pltpu.async_copy(
                tr_hbm.at[b * C + c, pl.ds(h0, CHUNKR), :],
                tbuf[slot][c - 1], sems[slot]))
        return cps

    pending = {0: start(0, 0)}
    acc = (jnp.zeros((L,), jnp.float32), jnp.zeros((L,), jnp.float32))
    for k in range(NCHUNK):
        slot = k % 2
        if k + 1 < NCHUNK:
            pending[k + 1] = start(k + 1, (k + 1) % 2)
        for cp in pending.pop(k):
            cp.wait()
        l0r, l1r, l2r, l3r = lbuf[slot]
        t1r, t2r, t3r = tbuf[slot]

        def lane_step(r, sl, acc, l0r=l0r, l1r=l1r, l2r=l2r, l3r=l3r,
                      t1r=t1r, t2r=t2r, t3r=t3r):
            l0 = l0r[r, sl]
            l1 = l1r[r, sl]
            l2 = l2r[r, sl]
            l3 = l3r[r, sl]
            t1 = t1r[r, sl]
            t2 = t2r[r, sl]
            t3 = t3r[r, sl]
            lsel = jnp.where(t1 == 1, l1, l0)
            lsel = jnp.where(t2 == 1, l2, lsel)
            lsel = jnp.where(t3 == 1, l3, lsel)
            sm = (l0 + l1) + (l2 + l3)
            pt = lsel * K_SEL + sm * K_SUM + SMOOTH
            om = 1.0 - pt
            return acc - (om * om) * _fast_log(pt)

        @plsc.parallel_loop(0, NVEC // 2, unroll=4, carry=acc)
        def acc(i, accs):
            acc_a, acc_b = accs
            r = jnp.right_shift(i, 4)
            co = jnp.left_shift(jnp.bitwise_and(i, 15), 5)
            co = pl.multiple_of(co, 2 * L)
            acc_a = lane_step(r, pl.ds(co, L), acc_a)
            acc_b = lane_step(r, pl.ds(pl.multiple_of(co + L, L), L), acc_b)
            return acc_a, acc_b

    accbuf[...] = acc[0] + acc[1]
    pltpu.sync_copy(accbuf, out_hbm.at[pl.ds(wid * L, L)])


def _tc_body(lref, t1ref, t2ref, t3ref, oref):
    l0 = lref[0]
    l1 = lref[1]
    l2 = lref[2]
    l3 = lref[3]
    t1 = t1ref[0]
    t2 = t2ref[0]
    t3 = t3ref[0]
    lsel = jnp.where(t1 == 1, l1, l0)
    lsel = jnp.where(t2 == 1, l2, lsel)
    lsel = jnp.where(t3 == 1, l3, lsel)
    sm = (l0 + l1) + (l2 + l3)
    pt = lsel * K_SEL + sm * K_SUM + SMOOTH
    om = 1.0 - pt
    loss = (om * om) * jnp.log(pt)
    s = -jnp.sum(loss)
    first = jnp.logical_and(pl.program_id(0) == 0, pl.program_id(1) == 0)
    oref[0, 0] = jnp.where(first, s, oref[0, 0] + s)


@functools.partial(jax.jit, static_argnames=())
def kernel(logit, target):
    lr = logit.reshape(B * C, H, W)
    tr = target.astype(jnp.int32).reshape(B * C, H, W)

    mesh = plsc.VectorSubcoreMesh(core_axis_name="c", subcore_axis_name="s")
    sc_fn = pl.kernel(
        _sc_body,
        mesh=mesh,
        out_type=jax.ShapeDtypeStruct((NW * L,), jnp.float32),
        scratch_types=(
            [pltpu.VMEM((CHUNKR, W), jnp.float32) for _ in range(2 * C)]
            + [pltpu.VMEM((CHUNKR, W), jnp.int32) for _ in range(2 * (C - 1))]
            + [pltpu.VMEM((L,), jnp.float32),
               pltpu.SemaphoreType.DMA,
               pltpu.SemaphoreType.DMA]
        ),
        compiler_params=pltpu.CompilerParams(use_tc_tiling_on_sc=True),
    )
    sc_partials = sc_fn(lr, tr)

    tc_partials = pl.pallas_call(
        _tc_body,
        grid=(TB_TC, H // HB_TC),
        in_specs=[
            pl.BlockSpec((C, HB_TC, W), lambda b, j: (SB_SC + b, j, 0)),
            pl.BlockSpec((1, HB_TC, W),
                         lambda b, j: (C * (SB_SC + b) + 1, j, 0)),
            pl.BlockSpec((1, HB_TC, W),
                         lambda b, j: (C * (SB_SC + b) + 2, j, 0)),
            pl.BlockSpec((1, HB_TC, W),
                         lambda b, j: (C * (SB_SC + b) + 3, j, 0)),
        ],
        out_specs=pl.BlockSpec(memory_space=pltpu.SMEM),
        out_shape=jax.ShapeDtypeStruct((1, 1), jnp.float32),
        compiler_params=pltpu.CompilerParams(
            dimension_semantics=("arbitrary", "arbitrary")),
    )(lr, tr, tr, tr)

    return (jnp.sum(sc_partials) + tc_partials[0, 0]) / N


# speedup vs baseline: 1.0666x; 1.0666x over previous
import functools

import jax
import jax.numpy as jnp
from jax import lax
from jax.experimental import pallas as pl
from jax.experimental.pallas import tpu as pltpu
from jax.experimental.pallas import tpu_sc as plsc

B, C, H, W = 8, 4, 512, 512
HW = H * W
N = B * HW
GAMMA = 2.0
SMOOTH = 1e-05

SB_SC = 3
TB_TC = B - SB_SC

NC, NS, L = 2, 16, 16
NW = NC * NS
ROWS_W = SB_SC * H // NW
CHUNKR = 16
NCHUNK = ROWS_W // CHUNKR
NVEC = CHUNKR * W // L

HB_TC = 128
LN2 = 0.6931471805599453
K_SEL = 1.0 - SMOOTH - SMOOTH / (C - 1)
K_SUM = SMOOTH / (C - 1)


def _fast_log(x):
    bits = lax.bitcast_convert_type(x, jnp.int32)
    ef = jnp.right_shift(bits, 23).astype(jnp.float32)
    mbits = jnp.bitwise_or(jnp.bitwise_and(bits, 0x007FFFFF), 0x3F800000)
    m = lax.bitcast_convert_type(mbits, jnp.float32)
    z = (m - 1.0) / (m + 1.0)
    z2 = z * z
    p = z * (2.0 + z2 * (2.0 / 3.0 + z2 * (2.0 / 5.0 + z2 * (2.0 / 7.0))))
    return ef * LN2 + (p - 127.0 * LN2)


def _sc_body(lr_hbm, tr_hbm, out_hbm,
             lb0_0, lb0_1, lb0_2, lb0_3, lb1_0, lb1_1, lb1_2, lb1_3,
             tb0_1, tb0_2, tb0_3, tb1_1, tb1_2, tb1_3,
             accbuf, sem0, sem1):
    cid = lax.axis_index("c")
    sid = lax.axis_index("s")
    wid = cid * NS + sid
    lbuf = ((lb0_0, lb0_1, lb0_2, lb0_3), (lb1_0, lb1_1, lb1_2, lb1_3))
    tbuf = ((tb0_1, tb0_2, tb0_3), (tb1_1, tb1_2, tb1_3))
    sems = (sem0, sem1)

    def start(k, slot):
        g0 = wid * ROWS_W + k * CHUNKR
        b = g0 // H
        h0 = g0 % H
        cps = []
        for c in range(C):
            cps.append(pltpu.async_copy(
                lr_hbm.at[b * C + c, pl.ds(h0, CHUNKR), :],
                lbuf[slot][c], sems[slot]))
        for c in range(1, C):
            cps.append(pltpu.async_copy(
                tr_hbm.at[b * C + c, pl.ds(h0, CHUNKR), :],
                tbuf[slot][c - 1], sems[slot]))
        return cps

    pending = {0: start(0, 0)}
    acc = (jnp.zeros((L,), jnp.float32), jnp.zeros((L,), jnp.float32))
    for k in range(NCHUNK):
        slot = k % 2
        if k + 1 < NCHUNK:
            pending[k + 1] = start(k + 1, (k + 1) % 2)
        for cp in pending.pop(k):
            cp.wait()
        l0r, l1r, l2r, l3r = lbuf[slot]
        t1r, t2r, t3r = tbuf[slot]

        def lane_step(r, sl, acc, l0r=l0r, l1r=l1r, l2r=l2r, l3r=l3r,
                      t1r=t1r, t2r=t2r, t3r=t3r):
            l0 = l0r[r, sl]
            l1 = l1r[r, sl]
            l2 = l2r[r, sl]
            l3 = l3r[r, sl]
            t1 = t1r[r, sl]
            t2 = t2r[r, sl]
            t3 = t3r[r, sl]
            lsel = jnp.where(t1 == 1, l1, l0)
            lsel = jnp.where(t2 == 1, l2, lsel)
            lsel = jnp.where(t3 == 1, l3, lsel)
            sm = (l0 + l1) + (l2 + l3)
            pt = lsel * K_SEL + sm * K_SUM + SMOOTH
            om = 1.0 - pt
            return acc - (om * om) * _fast_log(pt)

        @plsc.parallel_loop(0, NVEC // 2, unroll=4, carry=acc)
        def acc(i, accs):
            acc_a, acc_b = accs
            r = jnp.right_shift(i, 4)
            co = jnp.left_shift(jnp.bitwise_and(i, 15), 5)
            co = pl.multiple_of(co, 2 * L)
            acc_a = lane_step(r, pl.ds(co, L), acc_a)
            acc_b = lane_step(r, pl.ds(pl.multiple_of(co + L, L), L), acc_b)
            return acc_a, acc_b

    accbuf[...] = acc[0] + acc[1]
    pltpu.sync_copy(accbuf, out_hbm.at[pl.ds(wid * L, L)])


def _tc_body(lref, t1ref, t2ref, t3ref, oref):
    l0 = lref[0]
    l1 = lref[1]
    l2 = lref[2]
    l3 = lref[3]
    t1 = t1ref[0]
    t2 = t2ref[0]
    t3 = t3ref[0]
    lsel = jnp.where(t1 == 1, l1, l0)
    lsel = jnp.where(t2 == 1, l2, lsel)
    lsel = jnp.where(t3 == 1, l3, lsel)
    sm = (l0 + l1) + (l2 + l3)
    pt = lsel * K_SEL + sm * K_SUM + SMOOTH
    om = 1.0 - pt
    loss = (om * om) * jnp.log(pt)
    s = -jnp.sum(loss)
    first = jnp.logical_and(pl.program_id(0) == 0, pl.program_id(1) == 0)
    oref[0, 0] = jnp.where(first, s, oref[0, 0] + s)


@functools.partial(jax.jit, static_argnames=())
def kernel(logit, target):
    lr = logit.reshape(B * C, H, W)
    tr = target.astype(jnp.int32).reshape(B * C, H, W)

    mesh = plsc.VectorSubcoreMesh(core_axis_name="c", subcore_axis_name="s")
    sc_fn = pl.kernel(
        _sc_body,
        mesh=mesh,
        out_type=jax.ShapeDtypeStruct((NW * L,), jnp.float32),
        scratch_types=(
            [pltpu.VMEM((CHUNKR, W), jnp.float32) for _ in range(2 * C)]
            + [pltpu.VMEM((CHUNKR, W), jnp.int32) for _ in range(2 * (C - 1))]
            + [pltpu.VMEM((L,), jnp.float32),
               pltpu.SemaphoreType.DMA,
               pltpu.SemaphoreType.DMA]
        ),
        compiler_params=pltpu.CompilerParams(use_tc_tiling_on_sc=True),
    )
    sc_partials = sc_fn(lr, tr)

    tc_partials = pl.pallas_call(
        _tc_body,
        grid=(TB_TC, H // HB_TC),
        in_specs=[
            pl.BlockSpec((C, HB_TC, W), lambda b, j: (SB_SC + b, j, 0)),
            pl.BlockSpec((1, HB_TC, W),
                         lambda b, j: (C * (SB_SC + b) + 1, j, 0)),
            pl.BlockSpec((1, HB_TC, W),
                         lambda b, j: (C * (SB_SC + b) + 2, j, 0)),
            pl.BlockSpec((1, HB_TC, W),
                         lambda b, j: (C * (SB_SC + b) + 3, j, 0)),
        ],
        out_specs=pl.BlockSpec(memory_space=pltpu.SMEM),
        out_shape=jax.ShapeDtypeStruct((1, 1), jnp.float32),
        compiler_params=pltpu.CompilerParams(
            dimension_semantics=("arbitrary", "arbitrary")),
    )(lr, tr, tr, tr)

    return (jnp.sum(sc_partials) + tc_partials[0, 0]) / N


# speedup vs baseline: 1.0669x; 1.0003x over previous
import functools

import jax
import jax.numpy as jnp
from jax import lax
from jax.experimental import pallas as pl
from jax.experimental.pallas import tpu as pltpu
from jax.experimental.pallas import tpu_sc as plsc

B, C, H, W = 8, 4, 512, 512
HW = H * W
N = B * HW
GAMMA = 2.0
SMOOTH = 1e-05

SB_SC = 3
TB_TC = B - SB_SC

NC, NS, L = 2, 16, 16
NW = NC * NS
ROWS_W = SB_SC * H // NW
CHUNKR = 16
NCHUNK = ROWS_W // CHUNKR
NVEC = CHUNKR * W // L

HB_TC = 256
LN2 = 0.6931471805599453
K_SEL = 1.0 - SMOOTH - SMOOTH / (C - 1)
K_SUM = SMOOTH / (C - 1)


def _fast_log(x):
    bits = lax.bitcast_convert_type(x, jnp.int32)
    ef = jnp.right_shift(bits, 23).astype(jnp.float32)
    mbits = jnp.bitwise_or(jnp.bitwise_and(bits, 0x007FFFFF), 0x3F800000)
    m = lax.bitcast_convert_type(mbits, jnp.float32)
    z = (m - 1.0) / (m + 1.0)
    z2 = z * z
    p = z * (2.0 + z2 * (2.0 / 3.0 + z2 * (2.0 / 5.0 + z2 * (2.0 / 7.0))))
    return ef * LN2 + (p - 127.0 * LN2)


def _sc_body(lr_hbm, tr_hbm, out_hbm,
             lb0_0, lb0_1, lb0_2, lb0_3, lb1_0, lb1_1, lb1_2, lb1_3,
             tb0_1, tb0_2, tb0_3, tb1_1, tb1_2, tb1_3,
             accbuf, sem0, sem1):
    cid = lax.axis_index("c")
    sid = lax.axis_index("s")
    wid = cid * NS + sid
    lbuf = ((lb0_0, lb0_1, lb0_2, lb0_3), (lb1_0, lb1_1, lb1_2, lb1_3))
    tbuf = ((tb0_1, tb0_2, tb0_3), (tb1_1, tb1_2, tb1_3))
    sems = (sem0, sem1)

    def start(k, slot):
        g0 = wid * ROWS_W + k * CHUNKR
        b = g0 // H
        h0 = g0 % H
        cps = []
        for c in range(C):
            cps.append(pltpu.async_copy(
                lr_hbm.at[b * C + c, pl.ds(h0, CHUNKR), :],
                lbuf[slot][c], sems[slot]))
        for c in range(1, C):
            cps.append(pltpu.async_copy(
                tr_hbm.at[b * C + c, pl.ds(h0, CHUNKR), :],
                tbuf[slot][c - 1], sems[slot]))
        return cps

    pending = {0: start(0, 0)}
    acc = (jnp.zeros((L,), jnp.float32), jnp.zeros((L,), jnp.float32))
    for k in range(NCHUNK):
        slot = k % 2
        if k + 1 < NCHUNK:
            pending[k + 1] = start(k + 1, (k + 1) % 2)
        for cp in pending.pop(k):
            cp.wait()
        l0r, l1r, l2r, l3r = lbuf[slot]
        t1r, t2r, t3r = tbuf[slot]

        def lane_step(r, sl, acc, l0r=l0r, l1r=l1r, l2r=l2r, l3r=l3r,
                      t1r=t1r, t2r=t2r, t3r=t3r):
            l0 = l0r[r, sl]
            l1 = l1r[r, sl]
            l2 = l2r[r, sl]
            l3 = l3r[r, sl]
            t1 = t1r[r, sl]
            t2 = t2r[r, sl]
            t3 = t3r[r, sl]
            lsel = jnp.where(t1 == 1, l1, l0)
            lsel = jnp.where(t2 == 1, l2, lsel)
            lsel = jnp.where(t3 == 1, l3, lsel)
            sm = (l0 + l1) + (l2 + l3)
            pt = lsel * K_SEL + sm * K_SUM + SMOOTH
            om = 1.0 - pt
            return acc - (om * om) * _fast_log(pt)

        @plsc.parallel_loop(0, NVEC // 2, unroll=4, carry=acc)
        def acc(i, accs):
            acc_a, acc_b = accs
            r = jnp.right_shift(i, 4)
            co = jnp.left_shift(jnp.bitwise_and(i, 15), 5)
            co = pl.multiple_of(co, 2 * L)
            acc_a = lane_step(r, pl.ds(co, L), acc_a)
            acc_b = lane_step(r, pl.ds(pl.multiple_of(co + L, L), L), acc_b)
            return acc_a, acc_b

    accbuf[...] = acc[0] + acc[1]
    pltpu.sync_copy(accbuf, out_hbm.at[pl.ds(wid * L, L)])


def _tc_body(lref, t1ref, t2ref, t3ref, oref):
    l0 = lref[0]
    l1 = lref[1]
    l2 = lref[2]
    l3 = lref[3]
    t1 = t1ref[0]
    t2 = t2ref[0]
    t3 = t3ref[0]
    lsel = jnp.where(t1 == 1, l1, l0)
    lsel = jnp.where(t2 == 1, l2, lsel)
    lsel = jnp.where(t3 == 1, l3, lsel)
    sm = (l0 + l1) + (l2 + l3)
    pt = lsel * K_SEL + sm * K_SUM + SMOOTH
    om = 1.0 - pt
    loss = (om * om) * jnp.log(pt)
    s = -jnp.sum(loss)
    first = jnp.logical_and(pl.program_id(0) == 0, pl.program_id(1) == 0)
    oref[0, 0] = jnp.where(first, s, oref[0, 0] + s)


@functools.partial(jax.jit, static_argnames=())
def kernel(logit, target):
    lr = logit.reshape(B * C, H, W)
    tr = target.astype(jnp.int32).reshape(B * C, H, W)

    mesh = plsc.VectorSubcoreMesh(core_axis_name="c", subcore_axis_name="s")
    sc_fn = pl.kernel(
        _sc_body,
        mesh=mesh,
        out_type=jax.ShapeDtypeStruct((NW * L,), jnp.float32),
        scratch_types=(
            [pltpu.VMEM((CHUNKR, W), jnp.float32) for _ in range(2 * C)]
            + [pltpu.VMEM((CHUNKR, W), jnp.int32) for _ in range(2 * (C - 1))]
            + [pltpu.VMEM((L,), jnp.float32),
               pltpu.SemaphoreType.DMA,
               pltpu.SemaphoreType.DMA]
        ),
        compiler_params=pltpu.CompilerParams(use_tc_tiling_on_sc=True),
    )
    sc_partials = sc_fn(lr, tr)

    tc_partials = pl.pallas_call(
        _tc_body,
        grid=(TB_TC, H // HB_TC),
        in_specs=[
            pl.BlockSpec((C, HB_TC, W), lambda b, j: (SB_SC + b, j, 0)),
            pl.BlockSpec((1, HB_TC, W),
                         lambda b, j: (C * (SB_SC + b) + 1, j, 0)),
            pl.BlockSpec((1, HB_TC, W),
                         lambda b, j: (C * (SB_SC + b) + 2, j, 0)),
            pl.BlockSpec((1, HB_TC, W),
                         lambda b, j: (C * (SB_SC + b) + 3, j, 0)),
        ],
        out_specs=pl.BlockSpec(memory_space=pltpu.SMEM),
        out_shape=jax.ShapeDtypeStruct((1, 1), jnp.float32),
        compiler_params=pltpu.CompilerParams(
            dimension_semantics=("arbitrary", "arbitrary")),
    )(lr, tr, tr, tr)

    return (jnp.sum(sc_partials) + tc_partials[0, 0]) / N
